# Initial kernel scaffold; baseline (speedup 1.0000x reference)
import jax
import jax.numpy as jnp
from jax.experimental import pallas as pl


def kernel(x, x_mask, edge_index, W0, b0, W1, b1, W2, b2, W3, b3, Wr1, br1, Wr2, br2):
    raise NotImplementedError("write your pallas kernel here")


# speedup vs baseline: 16.6381x; 16.6381x over previous
import functools

import jax
import jax.numpy as jnp
from jax import lax
from jax.experimental import pallas as pl
from jax.experimental.pallas import tpu as pltpu
from jax.experimental.pallas import tpu_sc as plsc

N = 10000
F = 64
H = 128
NC = 2
NS = 16
NW = NC * NS
CHUNK = 128
E = 320000
CPT = 80
EPT = CPT * CHUNK
EPAD = NW * EPT
NPAD = 10240
RPT = NPAD // NS
WB = RPT // CHUNK

_mesh = plsc.VectorSubcoreMesh(core_axis_name="c", subcore_axis_name="s")


@functools.partial(
    pl.kernel,
    mesh=_mesh,
    out_type=jax.ShapeDtypeStruct((NC * NPAD,), jnp.float32),
    scratch_types=[
        pltpu.VMEM((CPT, CHUNK), jnp.int32),
        pltpu.VMEM((CHUNK,), jnp.float32),
        pltpu.VMEM((RPT,), jnp.float32),
        pltpu.VMEM_SHARED((NPAD,), jnp.float32),
    ],
)
def _sc_degree(dst_hbm, deg_hbm, dstv, ones_v, zbuf, acc):
    cid = lax.axis_index("c")
    sid = lax.axis_index("s")
    wid = sid * NC + cid
    pltpu.sync_copy(dst_hbm.at[pl.ds(wid * CPT, CPT)], dstv)
    for i in range(CHUNK // 16):
        ones_v[pl.ds(i * 16, 16)] = jnp.ones((16,), jnp.float32)

    def zbody(i, carry):
        zbuf[pl.ds(i * 16, 16)] = jnp.zeros((16,), jnp.float32)
        return carry

    lax.fori_loop(0, RPT // 16, zbody, 0)
    pltpu.sync_copy(zbuf, acc.at[pl.ds(sid * RPT, RPT)])
    plsc.subcore_barrier()

    def body(j, carry):
        pltpu.sync_copy(ones_v, acc.at[dstv.at[j]], add=True)
        return carry

    lax.fori_loop(0, CPT, body, 0)
    plsc.subcore_barrier()
    pltpu.sync_copy(acc.at[pl.ds(sid * RPT, RPT)], zbuf)
    pltpu.sync_copy(zbuf, deg_hbm.at[pl.ds(cid * NPAD + sid * RPT, RPT)])


@functools.partial(
    pl.kernel,
    mesh=_mesh,
    out_type=jax.ShapeDtypeStruct((NC, NPAD, H), jnp.float32),
    scratch_types=[
        pltpu.VMEM((CPT, CHUNK), jnp.int32),
        pltpu.VMEM((CPT, CHUNK), jnp.int32),
        pltpu.VMEM((CHUNK, H), jnp.float32),
        pltpu.VMEM_SHARED((NPAD, H), jnp.float32),
        pltpu.SemaphoreType.DMA,
    ],
)
def _sc_spmm(g_hbm, src_hbm, dst_hbm, out_hbm,
             srcv, dstv, rows, acc, sem):
    cid = lax.axis_index("c")
    sid = lax.axis_index("s")
    wid = sid * NC + cid
    pltpu.sync_copy(src_hbm.at[pl.ds(wid * CPT, CPT)], srcv)
    pltpu.sync_copy(dst_hbm.at[pl.ds(wid * CPT, CPT)], dstv)

    def zbody(i, carry):
        for k in range(H // 16):
            rows[i, pl.ds(k * 16, 16)] = jnp.zeros((16,), jnp.float32)
        return carry

    lax.fori_loop(0, CHUNK, zbody, 0)
    for k in range(WB):
        pltpu.sync_copy(rows, acc.at[pl.ds(sid * RPT + k * CHUNK, CHUNK)])
    plsc.subcore_barrier()

    def body(j, carry):
        pltpu.async_copy(g_hbm.at[srcv.at[j]], rows, sem).wait()
        pltpu.sync_copy(rows, acc.at[dstv.at[j]], add=True)
        return carry

    lax.fori_loop(0, CPT, body, 0)
    plsc.subcore_barrier()
    for k in range(WB):
        pltpu.sync_copy(acc.at[pl.ds(sid * RPT + k * CHUNK, CHUNK)], rows)
        pltpu.sync_copy(rows, out_hbm.at[cid, pl.ds(sid * RPT + k * CHUNK, CHUNK)])


def _tc_first_body(x_ref, xm_ref, w0a_ref, w0b_ref, degp_ref, g_ref, dinv_ref):
    deg = 1.0 + degp_ref[0, :N, :] + degp_ref[1, :N, :]
    dinv = lax.rsqrt(deg)
    z = (jnp.dot(x_ref[...], w0a_ref[...], preferred_element_type=jnp.float32)
         + jnp.dot(xm_ref[...], w0b_ref[...], preferred_element_type=jnp.float32))
    dinv_ref[...] = dinv
    g_ref[...] = dinv * z


_tc_first = pl.pallas_call(
    _tc_first_body,
    out_shape=(jax.ShapeDtypeStruct((N, H), jnp.float32),
               jax.ShapeDtypeStruct((N, 1), jnp.float32)),
)


def _tc_mid_body(p_ref, g_ref, dinv_ref, b_ref, w_ref, gout_ref):
    dinv = dinv_ref[...]
    agg = p_ref[0, :N, :] + p_ref[1, :N, :] + g_ref[...]
    h = jnp.maximum(dinv * agg + b_ref[...], 0.0)
    gout_ref[...] = dinv * jnp.dot(h, w_ref[...],
                                   preferred_element_type=jnp.float32)


_tc_mid = pl.pallas_call(
    _tc_mid_body,
    out_shape=jax.ShapeDtypeStruct((N, H), jnp.float32),
)


def _tc_last_body(p_ref, g_ref, dinv_ref, b_ref, wr1_ref, br1_ref,
                  wr2r_ref, br2_ref, emb_ref, pred_ref):
    dinv = dinv_ref[...]
    emb = dinv * (p_ref[0, :N, :] + p_ref[1, :N, :] + g_ref[...]) + b_ref[...]
    emb_ref[...] = emb
    h = jnp.maximum(emb, 0.0)
    t = jnp.maximum(jnp.dot(h, wr1_ref[...], preferred_element_type=jnp.float32)
                    + br1_ref[...], 0.0)
    pred_ref[...] = jnp.sum(t * wr2r_ref[...], axis=1, keepdims=True) + br2_ref[...]


_tc_last = pl.pallas_call(
    _tc_last_body,
    out_shape=(jax.ShapeDtypeStruct((N, H), jnp.float32),
               jax.ShapeDtypeStruct((N, 1), jnp.float32)),
)


def kernel(x, x_mask, edge_index, W0, b0, W1, b1, W2, b2, W3, b3,
           Wr1, br1, Wr2, br2):
    src = edge_index[0].astype(jnp.int32)
    dst = edge_index[1].astype(jnp.int32)
    pad = EPAD - E
    ar = jnp.arange(pad, dtype=jnp.int32)
    src_p = jnp.concatenate([src, (ar * 97) % N]).reshape(NW * CPT, CHUNK)
    dst_p = jnp.concatenate([dst, N + (ar % (NPAD - N))]).reshape(NW * CPT, CHUNK)

    degp = _sc_degree(dst_p).reshape(NC, NPAD, 1)
    g0, dinv = _tc_first(x[:, :F], x_mask[:, :F], W0[:F], W0[F:], degp)
    p = _sc_spmm(g0, src_p, dst_p)
    g1 = _tc_mid(p, g0, dinv, b0.reshape(1, H), W1)
    p = _sc_spmm(g1, src_p, dst_p)
    g2 = _tc_mid(p, g1, dinv, b1.reshape(1, H), W2)
    p = _sc_spmm(g2, src_p, dst_p)
    g3 = _tc_mid(p, g2, dinv, b2.reshape(1, H), W3)
    p = _sc_spmm(g3, src_p, dst_p)
    emb, pred = _tc_last(p, g3, dinv, b3.reshape(1, H), Wr1,
                         br1.reshape(1, H), Wr2.reshape(1, H),
                         br2.reshape(1, 1))
    return emb, pred


# speedup vs baseline: 25.2916x; 1.5201x over previous
import functools

import jax
import jax.numpy as jnp
from jax import lax
from jax.experimental import pallas as pl
from jax.experimental.pallas import tpu as pltpu
from jax.experimental.pallas import tpu_sc as plsc

N = 10000
F = 64
H = 128
NC = 2
NS = 16
NW = NC * NS
CHUNK = 128
E = 320000
CPT = 80
EPT = CPT * CHUNK
EPAD = NW * EPT
NPAD = 10240
RPT = NPAD // NS
WB = RPT // CHUNK
NBUF = 2
PKM = 16383

_mesh = plsc.VectorSubcoreMesh(core_axis_name="c", subcore_axis_name="s")


@functools.partial(
    pl.kernel,
    mesh=_mesh,
    out_type=jax.ShapeDtypeStruct((NC * NPAD,), jnp.float32),
    scratch_types=[
        pltpu.VMEM((CPT, CHUNK), jnp.int32),
        pltpu.VMEM((CHUNK,), jnp.float32),
        pltpu.VMEM((RPT,), jnp.float32),
        pltpu.VMEM_SHARED((NPAD,), jnp.float32),
    ],
)
def _sc_degree(dst_hbm, deg_hbm, dstv, ones_v, zbuf, acc):
    cid = lax.axis_index("c")
    sid = lax.axis_index("s")
    wid = sid * NC + cid
    pltpu.sync_copy(dst_hbm.at[pl.ds(wid * CPT, CPT)], dstv)
    for i in range(CHUNK // 16):
        ones_v[pl.ds(i * 16, 16)] = jnp.ones((16,), jnp.float32)

    def zbody(i, carry):
        zbuf[pl.ds(i * 16, 16)] = jnp.zeros((16,), jnp.float32)
        return carry

    lax.fori_loop(0, RPT // 16, zbody, 0)
    pltpu.sync_copy(zbuf, acc.at[pl.ds(sid * RPT, RPT)])
    plsc.subcore_barrier()

    def body(j, carry):
        pltpu.sync_copy(ones_v, acc.at[dstv.at[j]], add=True)
        return carry

    lax.fori_loop(0, CPT, body, 0)
    plsc.subcore_barrier()
    pltpu.sync_copy(acc.at[pl.ds(sid * RPT, RPT)], zbuf)
    pltpu.sync_copy(zbuf, deg_hbm.at[pl.ds(cid * NPAD + sid * RPT, RPT)])


@functools.partial(
    pl.kernel,
    mesh=_mesh,
    out_type=jax.ShapeDtypeStruct((NC, NPAD, H), jnp.float32),
    scratch_types=[
        pltpu.VMEM((CPT, CHUNK), jnp.int32),
    ]
    + [pltpu.VMEM((CHUNK,), jnp.int32) for _ in range(NBUF)]
    + [pltpu.VMEM((CHUNK,), jnp.int32) for _ in range(NBUF)]
    + [pltpu.VMEM((CHUNK, H), jnp.float32) for _ in range(NBUF)]
    + [pltpu.VMEM_SHARED((NPAD, H), jnp.float32)]
    + [pltpu.SemaphoreType.DMA for _ in range(NBUF)],
)
def _sc_spmm(g_hbm, pk_hbm, out_hbm, pk, *rest):
    sidx = rest[:NBUF]
    didx = rest[NBUF:2 * NBUF]
    rows = rest[2 * NBUF:3 * NBUF]
    acc = rest[3 * NBUF]
    sems = rest[3 * NBUF + 1:]
    cid = lax.axis_index("c")
    sid = lax.axis_index("s")
    wid = sid * NC + cid
    pltpu.sync_copy(pk_hbm.at[pl.ds(wid * CPT, CPT)], pk)

    def zbody(i, carry):
        for k in range(H // 16):
            rows[0][i, pl.ds(k * 16, 16)] = jnp.zeros((16,), jnp.float32)
        return carry

    lax.fori_loop(0, CHUNK, zbody, 0)
    for k in range(WB):
        pltpu.sync_copy(rows[0], acc.at[pl.ds(sid * RPT + k * CHUNK, CHUNK)])
    plsc.subcore_barrier()

    def unpack(j, b):
        for k in range(CHUNK // 16):
            pe = pk[j, pl.ds(k * 16, 16)]
            sidx[b][pl.ds(k * 16, 16)] = lax.bitwise_and(pe, PKM)
            didx[b][pl.ds(k * 16, 16)] = lax.shift_right_logical(pe, 14)

    for b in range(NBUF):
        unpack(b, b)
        pltpu.async_copy(g_hbm.at[sidx[b]], rows[b], sems[b])

    def body(i, carry):
        for b in range(NBUF):
            j = i * NBUF + b
            pltpu.make_async_copy(g_hbm.at[sidx[b]], rows[b], sems[b]).wait()
            pltpu.sync_copy(rows[b], acc.at[didx[b]], add=True)
            unpack(j + NBUF, b)
            pltpu.async_copy(g_hbm.at[sidx[b]], rows[b], sems[b])
        return carry

    lax.fori_loop(0, CPT // NBUF - 1, body, 0)
    for b in range(NBUF):
        pltpu.make_async_copy(g_hbm.at[sidx[b]], rows[b], sems[b]).wait()
        pltpu.sync_copy(rows[b], acc.at[didx[b]], add=True)
    plsc.subcore_barrier()
    for k in range(WB):
        pltpu.sync_copy(acc.at[pl.ds(sid * RPT + k * CHUNK, CHUNK)], rows[0])
        pltpu.sync_copy(rows[0], out_hbm.at[cid, pl.ds(sid * RPT + k * CHUNK, CHUNK)])


def _tc_first_body(x_ref, xm_ref, w0a_ref, w0b_ref, degp_ref, g_ref, dinv_ref):
    deg = 1.0 + degp_ref[0, :N, :] + degp_ref[1, :N, :]
    dinv = lax.rsqrt(deg)
    z = (jnp.dot(x_ref[...], w0a_ref[...], preferred_element_type=jnp.float32)
         + jnp.dot(xm_ref[...], w0b_ref[...], preferred_element_type=jnp.float32))
    dinv_ref[...] = dinv
    g_ref[...] = dinv * z


_tc_first = pl.pallas_call(
    _tc_first_body,
    out_shape=(jax.ShapeDtypeStruct((N, H), jnp.float32),
               jax.ShapeDtypeStruct((N, 1), jnp.float32)),
)


def _tc_mid_body(p_ref, g_ref, dinv_ref, b_ref, w_ref, gout_ref):
    dinv = dinv_ref[...]
    agg = p_ref[0, :N, :] + p_ref[1, :N, :] + g_ref[...]
    h = jnp.maximum(dinv * agg + b_ref[...], 0.0)
    gout_ref[...] = dinv * jnp.dot(h, w_ref[...],
                                   preferred_element_type=jnp.float32)


_tc_mid = pl.pallas_call(
    _tc_mid_body,
    out_shape=jax.ShapeDtypeStruct((N, H), jnp.float32),
)


def _tc_last_body(p_ref, g_ref, dinv_ref, b_ref, wr1_ref, br1_ref,
                  wr2r_ref, br2_ref, emb_ref, pred_ref):
    dinv = dinv_ref[...]
    emb = dinv * (p_ref[0, :N, :] + p_ref[1, :N, :] + g_ref[...]) + b_ref[...]
    emb_ref[...] = emb
    h = jnp.maximum(emb, 0.0)
    t = jnp.maximum(jnp.dot(h, wr1_ref[...], preferred_element_type=jnp.float32)
                    + br1_ref[...], 0.0)
    pred_ref[...] = jnp.sum(t * wr2r_ref[...], axis=1, keepdims=True) + br2_ref[...]


_tc_last = pl.pallas_call(
    _tc_last_body,
    out_shape=(jax.ShapeDtypeStruct((N, H), jnp.float32),
               jax.ShapeDtypeStruct((N, 1), jnp.float32)),
)


def kernel(x, x_mask, edge_index, W0, b0, W1, b1, W2, b2, W3, b3,
           Wr1, br1, Wr2, br2):
    src = edge_index[0].astype(jnp.int32)
    dst = edge_index[1].astype(jnp.int32)
    pad = EPAD - E
    ar = jnp.arange(pad, dtype=jnp.int32)
    src_p = jnp.concatenate([src, (ar * 97) % N]).reshape(NW * CPT, CHUNK)
    dst_p = jnp.concatenate([dst, N + (ar % (NPAD - N))]).reshape(NW * CPT, CHUNK)
    pk_p = dst_p * (PKM + 1) + src_p

    degp = _sc_degree(dst_p).reshape(NC, NPAD, 1)
    g0, dinv = _tc_first(x[:, :F], x_mask[:, :F], W0[:F], W0[F:], degp)
    p = _sc_spmm(g0, pk_p)
    g1 = _tc_mid(p, g0, dinv, b0.reshape(1, H), W1)
    p = _sc_spmm(g1, pk_p)
    g2 = _tc_mid(p, g1, dinv, b1.reshape(1, H), W2)
    p = _sc_spmm(g2, pk_p)
    g3 = _tc_mid(p, g2, dinv, b2.reshape(1, H), W3)
    p = _sc_spmm(g3, pk_p)
    emb, pred = _tc_last(p, g3, dinv, b3.reshape(1, H), Wr1,
                         br1.reshape(1, H), Wr2.reshape(1, H),
                         br2.reshape(1, 1))
    return emb, pred
